# Initial kernel scaffold; baseline (speedup 1.0000x reference)
import jax
import jax.numpy as jnp
from jax.experimental import pallas as pl


def kernel(x, edge_index, W0, A0_w1, A0_b1, A0_w2, A0_b2, theta0, bias0, W1, A1_w1, A1_b1, A1_w2, A1_b2, theta1, bias1):
    raise NotImplementedError("write your pallas kernel here")


# speedup vs baseline: 4.7180x; 4.7180x over previous
import functools

import jax
import jax.numpy as jnp
from jax import lax
from jax.experimental import pallas as pl
from jax.experimental.pallas import tpu as pltpu
from jax.experimental.pallas import tpu_sc as plsc

N = 10000
E = 160000
D_IN = 128
H0, F0 = 4, 64
HID = 64

N_PAD = 10240
TRASH = N
NSUB = 32
EPW = 5120
E_PAD = NSUB * EPW
C = 128
NCH = EPW // C
EPS = E_PAD // 16
NCH2 = EPS // C
RPS = N_PAD // 16
BLK = 1024


def _mesh():
    return plsc.VectorSubcoreMesh(core_axis_name="c", subcore_axis_name="s")


_SC_PARAMS = pltpu.CompilerParams(needs_layout_passes=False)


def _prep_body(x_ref, w0_ref, w1_ref, b1_ref, proj_ref, ta_ref, tb_ref,
               tc_ref, p0_ref, p1_ref):
    proj = jnp.dot(x_ref[...], w0_ref[...], preferred_element_type=jnp.float32)
    proj_ref[...] = proj
    p0_ref[...] = proj[:, :128]
    p1_ref[...] = proj[:, 128:]
    w1a = w1_ref[0:F0, :]
    w1b = w1_ref[F0:2 * F0, :]
    w1c = w1_ref[2 * F0:3 * F0, :]
    for h in range(H0):
        ph = proj[:, h * F0:(h + 1) * F0]
        ta_ref[:, h * F0:(h + 1) * F0] = (
            jnp.dot(ph, w1a, preferred_element_type=jnp.float32) + b1_ref[...])
        tb_ref[:, h * F0:(h + 1) * F0] = jnp.dot(
            ph, w1b, preferred_element_type=jnp.float32)
        tc_ref[:, h * F0:(h + 1) * F0] = jnp.dot(
            ph, w1c, preferred_element_type=jnp.float32)


def _prep(x_pad, W0, A0_w1, b1row):
    grid = (N_PAD // BLK,)
    out = [jax.ShapeDtypeStruct((N_PAD, 256), jnp.float32)] * 4 + [
        jax.ShapeDtypeStruct((N_PAD, 128), jnp.float32)] * 2
    return pl.pallas_call(
        _prep_body,
        grid=grid,
        in_specs=[
            pl.BlockSpec((BLK, D_IN), lambda i: (i, 0)),
            pl.BlockSpec((D_IN, 256), lambda i: (0, 0)),
            pl.BlockSpec((3 * F0, HID), lambda i: (0, 0)),
            pl.BlockSpec((1, HID), lambda i: (0, 0)),
        ],
        out_specs=[pl.BlockSpec((BLK, 256), lambda i: (i, 0))] * 4
        + [pl.BlockSpec((BLK, 128), lambda i: (i, 0))] * 2,
        out_shape=out,
    )(x_pad, W0, A0_w1, b1row)


def _att_body(ta_hbm, tb_hbm, tc_hbm, ii_hbm, jj_hbm, kk_hbm, w2_hbm,
              a_hbm, ii_v, jj_v, kk_v, tab, tbb, tcb, a_buf, w2_v):
    cid = lax.axis_index("c")
    sid = lax.axis_index("s")
    wid = sid * 2 + cid
    pltpu.sync_copy(w2_hbm, w2_v)
    base_w = wid * EPW

    @pl.loop(0, NCH)
    def _chunk(ch):
        base = base_w + ch * C
        pltpu.sync_copy(ii_hbm.at[pl.ds(base, C)], ii_v)
        pltpu.sync_copy(jj_hbm.at[pl.ds(base, C)], jj_v)
        pltpu.sync_copy(kk_hbm.at[pl.ds(base, C)], kk_v)
        pltpu.sync_copy(ta_hbm.at[ii_v], tab)
        pltpu.sync_copy(tb_hbm.at[jj_v], tbb)
        pltpu.sync_copy(tc_hbm.at[kk_v], tcb)

        @pl.loop(0, C // 16)
        def _grp(g):
            r = lax.iota(jnp.int32, 16) + g * 16
            zero = jnp.zeros((16,), jnp.float32)

            def dstep(d, accs):
                w2d = w2_v[pl.ds(d, 16)][0]
                new = []
                for h in range(H0):
                    cidx = jnp.full((16,), h * F0, jnp.int32) + d
                    v = (plsc.load_gather(tab, [r, cidx])
                         + plsc.load_gather(tbb, [r, cidx])
                         + plsc.load_gather(tcb, [r, cidx]))
                    new.append(accs[h] + jnp.maximum(v, 0.0) * w2d)
                return tuple(new)

            accs = lax.fori_loop(0, HID, dstep, (zero, zero, zero, zero))
            b2 = w2_v[pl.ds(HID, 16)][0]
            ss = [a + b2 for a in accs]
            ss = [jnp.where(s > 0, s, 0.2 * s) for s in ss]
            m = jnp.maximum(jnp.maximum(ss[0], ss[1]),
                            jnp.maximum(ss[2], ss[3]))
            es = [jnp.exp(s - m) for s in ss]
            inv = 1.0 / (es[0] + es[1] + es[2] + es[3])
            for h in range(H0):
                a_buf[h, pl.ds(g * 16, 16)] = es[h] * inv

        pltpu.sync_copy(a_buf, a_hbm.at[:, pl.ds(base, C)])


def _attention(TA, TB, TCt, ii, jj, kk, w2b2):
    kfn = pl.kernel(
        _att_body,
        out_type=jax.ShapeDtypeStruct((H0, E_PAD), jnp.float32),
        mesh=_mesh(),
        scratch_types=[
            pltpu.VMEM((C,), jnp.int32),
            pltpu.VMEM((C,), jnp.int32),
            pltpu.VMEM((C,), jnp.int32),
            pltpu.VMEM((C, 256), jnp.float32),
            pltpu.VMEM((C, 256), jnp.float32),
            pltpu.VMEM((C, 256), jnp.float32),
            pltpu.VMEM((H0, C), jnp.float32),
            pltpu.VMEM((80,), jnp.float32),
        ],
        compiler_params=_SC_PARAMS,
    )
    return kfn(TA, TB, TCt, ii, jj, kk, w2b2)


def _agg0_body(p0_hbm, p1_hbm, ii_hbm, jj_hbm, kk_hbm, a_hbm, z_hbm,
               ws0_hbm, ws1_hbm, ii_v, jj_v, kk_v, pjb, pkb, a2, ws_sh):
    cid = lax.axis_index("c")
    sid = lax.axis_index("s")
    rows = pl.ds(sid * RPS, RPS)
    pltpu.sync_copy(z_hbm.at[rows], ws_sh.at[rows])
    plsc.subcore_barrier()
    base_s = sid * EPS

    @pl.loop(0, NCH2)
    def _chunk(ch):
        base = base_s + ch * C
        pltpu.sync_copy(ii_hbm.at[pl.ds(base, C)], ii_v)
        pltpu.sync_copy(jj_hbm.at[pl.ds(base, C)], jj_v)
        pltpu.sync_copy(kk_hbm.at[pl.ds(base, C)], kk_v)

        @pl.when(cid == 0)
        def _():
            pltpu.sync_copy(p0_hbm.at[jj_v], pjb)
            pltpu.sync_copy(p0_hbm.at[kk_v], pkb)
            pltpu.sync_copy(a_hbm.at[0, pl.ds(base, C)], a2.at[0, pl.ds(0, C)])
            pltpu.sync_copy(a_hbm.at[1, pl.ds(base, C)], a2.at[1, pl.ds(0, C)])

        @pl.when(cid == 1)
        def _():
            pltpu.sync_copy(p1_hbm.at[jj_v], pjb)
            pltpu.sync_copy(p1_hbm.at[kk_v], pkb)
            pltpu.sync_copy(a_hbm.at[2, pl.ds(base, C)], a2.at[0, pl.ds(0, C)])
            pltpu.sync_copy(a_hbm.at[3, pl.ds(base, C)], a2.at[1, pl.ds(0, C)])

        @pl.loop(0, C)
        def _edge(e):
            a0 = a2[0, pl.ds(e, 16)][0]
            a1 = a2[1, pl.ds(e, 16)][0]
            for v in range(8):
                sl = pl.ds(v * 16, 16)
                scale = a0 if v < 4 else a1
                pjb[e, sl] = pjb[e, sl] * pkb[e, sl] * scale

        pltpu.sync_copy(pjb, ws_sh.at[ii_v], add=True)

    plsc.subcore_barrier()

    @pl.when(cid == 0)
    def _():
        pltpu.sync_copy(ws_sh.at[rows], ws0_hbm.at[rows])

    @pl.when(cid == 1)
    def _():
        pltpu.sync_copy(ws_sh.at[rows], ws1_hbm.at[rows])


def _aggregate0(P0, P1, ii, jj, kk, a_att, zeros128):
    kfn = pl.kernel(
        _agg0_body,
        out_type=(jax.ShapeDtypeStruct((N_PAD, 128), jnp.float32),
                  jax.ShapeDtypeStruct((N_PAD, 128), jnp.float32)),
        mesh=_mesh(),
        scratch_types=[
            pltpu.VMEM((C,), jnp.int32),
            pltpu.VMEM((C,), jnp.int32),
            pltpu.VMEM((C,), jnp.int32),
            pltpu.VMEM((C, 128), jnp.float32),
            pltpu.VMEM((C, 128), jnp.float32),
            pltpu.VMEM((2, C + 16), jnp.float32),
            pltpu.VMEM_SHARED((N_PAD, 128), jnp.float32),
        ],
        compiler_params=_SC_PARAMS,
    )
    return kfn(P0, P1, ii, jj, kk, a_att, zeros128)


def _mid_body(proj_ref, ws0_ref, ws1_ref, th_ref, b0_ref, w1_ref, p1_ref):
    i = pl.program_id(0)
    ha = (th_ref[0, :128] * proj_ref[:, :128] + ws0_ref[...]
          + b0_ref[0, :128])
    hb = (th_ref[0, 128:] * proj_ref[:, 128:] + ws1_ref[...]
          + b0_ref[0, 128:])
    ha = jnp.where(ha > 0, ha, jnp.exp(jnp.minimum(ha, 0.0)) - 1.0)
    hb = jnp.where(hb > 0, hb, jnp.exp(jnp.minimum(hb, 0.0)) - 1.0)
    p1 = (jnp.dot(ha, w1_ref[:128, :], preferred_element_type=jnp.float32)
          + jnp.dot(hb, w1_ref[128:, :], preferred_element_type=jnp.float32))
    row = i * BLK + lax.broadcasted_iota(jnp.int32, (BLK, 1), 0)
    p1_ref[:, :64] = jnp.where(row < N, p1, 0.0)
    p1_ref[:, 64:] = jnp.zeros((BLK, 64), jnp.float32)


def _mid(proj0, ws0, ws1, th0row, b0row, W1):
    return pl.pallas_call(
        _mid_body,
        grid=(N_PAD // BLK,),
        in_specs=[
            pl.BlockSpec((BLK, 256), lambda i: (i, 0)),
            pl.BlockSpec((BLK, 128), lambda i: (i, 0)),
            pl.BlockSpec((BLK, 128), lambda i: (i, 0)),
            pl.BlockSpec((1, 256), lambda i: (0, 0)),
            pl.BlockSpec((1, 256), lambda i: (0, 0)),
            pl.BlockSpec((256, 64), lambda i: (0, 0)),
        ],
        out_specs=pl.BlockSpec((BLK, 128), lambda i: (i, 0)),
        out_shape=jax.ShapeDtypeStruct((N_PAD, 128), jnp.float32),
    )(proj0, ws0, ws1, th0row, b0row, W1)


def _agg1_body(pt_hbm, ii_hbm, jj_hbm, kk_hbm, z_hbm, wsp_hbm,
               ii_v, jj_v, kk_v, pjb, pkb, ws_sh):
    cid = lax.axis_index("c")
    sid = lax.axis_index("s")
    rows = pl.ds(sid * RPS, RPS)
    pltpu.sync_copy(z_hbm.at[rows], ws_sh.at[rows])
    plsc.subcore_barrier()
    wid = sid * 2 + cid
    base_w = wid * EPW

    @pl.loop(0, NCH)
    def _chunk(ch):
        base = base_w + ch * C
        pltpu.sync_copy(ii_hbm.at[pl.ds(base, C)], ii_v)
        pltpu.sync_copy(jj_hbm.at[pl.ds(base, C)], jj_v)
        pltpu.sync_copy(kk_hbm.at[pl.ds(base, C)], kk_v)
        pltpu.sync_copy(pt_hbm.at[jj_v], pjb)
        pltpu.sync_copy(pt_hbm.at[kk_v], pkb)

        @pl.loop(0, C)
        def _edge(e):
            for v in range(4):
                sl = pl.ds(v * 16, 16)
                pjb[e, sl] = pjb[e, sl] * pkb[e, sl]

        pltpu.sync_copy(pjb, ws_sh.at[ii_v], add=True)

    plsc.subcore_barrier()
    pltpu.sync_copy(ws_sh.at[rows], wsp_hbm.at[cid, rows])


def _aggregate1(Ptab, ii, jj, kk, zeros128):
    kfn = pl.kernel(
        _agg1_body,
        out_type=jax.ShapeDtypeStruct((2, N_PAD, 128), jnp.float32),
        mesh=_mesh(),
        scratch_types=[
            pltpu.VMEM((C,), jnp.int32),
            pltpu.VMEM((C,), jnp.int32),
            pltpu.VMEM((C,), jnp.int32),
            pltpu.VMEM((C, 128), jnp.float32),
            pltpu.VMEM((C, 128), jnp.float32),
            pltpu.VMEM_SHARED((N_PAD, 128), jnp.float32),
        ],
        compiler_params=_SC_PARAMS,
    )
    return kfn(Ptab, ii, jj, kk, zeros128)


def _fin_body(pt_ref, wsp_ref, th_ref, b1_ref, out_ref):
    p = pt_ref[:, :64]
    out_ref[...] = (th_ref[0][None] * p + wsp_ref[0, :, :64]
                    + wsp_ref[1, :, :64] + b1_ref[0][None])


def _fin(Ptab, wsp, th1row, b1row):
    FB = 1000
    return pl.pallas_call(
        _fin_body,
        grid=(N // FB,),
        in_specs=[
            pl.BlockSpec((FB, 128), lambda i: (i, 0)),
            pl.BlockSpec((2, FB, 128), lambda i: (0, i, 0)),
            pl.BlockSpec((1, 64), lambda i: (0, 0)),
            pl.BlockSpec((1, 64), lambda i: (0, 0)),
        ],
        out_specs=pl.BlockSpec((FB, 64), lambda i: (i, 0)),
        out_shape=jax.ShapeDtypeStruct((N, 64), jnp.float32),
    )(Ptab, wsp, th1row, b1row)


def kernel(x, edge_index, W0, A0_w1, A0_b1, A0_w2, A0_b2, theta0, bias0,
           W1, A1_w1, A1_b1, A1_w2, A1_b2, theta1, bias1):
    f32 = jnp.float32
    x_pad = jnp.zeros((N_PAD, D_IN), f32).at[:N].set(x)
    pad = jnp.full((E_PAD - E,), TRASH, jnp.int32)
    ii = jnp.concatenate([edge_index[0], pad])
    jj = jnp.concatenate([edge_index[1], pad])
    kk = jnp.concatenate([edge_index[2], pad])
    w2b2 = jnp.concatenate(
        [A0_w2[:, 0], A0_b2, jnp.zeros((15,), f32)])
    b1row = A0_b1.reshape(1, HID)
    th0row = theta0.reshape(1, H0 * F0)
    b0row = bias0.reshape(1, H0 * F0)
    th1row = theta1.reshape(1, 64)
    bias1row = bias1.reshape(1, 64)
    zeros128 = jnp.zeros((N_PAD, 128), f32)

    proj0, TA, TB, TCt, P0, P1 = _prep(x_pad, W0, A0_w1, b1row)
    a_att = _attention(TA, TB, TCt, ii, jj, kk, w2b2)
    ws0, ws1 = _aggregate0(P0, P1, ii, jj, kk, a_att, zeros128)
    Ptab = _mid(proj0, ws0, ws1, th0row, b0row, W1)
    wsp = _aggregate1(Ptab, ii, jj, kk, zeros128)
    return _fin(Ptab, wsp, th1row, bias1row)
